# Initial kernel scaffold; baseline (speedup 1.0000x reference)
import jax
import jax.numpy as jnp
from jax.experimental import pallas as pl


def kernel(tokens, table):
    raise NotImplementedError("write your pallas kernel here")


# speedup vs baseline: 7.8613x; 7.8613x over previous
import functools
import math

import jax
import jax.numpy as jnp
from jax import lax
from jax.experimental import pallas as pl
from jax.experimental.pallas import tpu as pltpu
from jax.experimental.pallas import tpu_sc as plsc

_V = 100000
_D = 128
_L = 16
_NC = 2
_NS = 16
_NW = _NC * _NS
_B = 1024 * 200
_BPW = _B // _NW
_CHUNK = 128
_NCH = _BPW // _CHUNK
_SCALE = math.sqrt(_D)

_mesh = plsc.VectorSubcoreMesh(
    core_axis_name="c", subcore_axis_name="s", num_cores=_NC, num_subcores=_NS
)


@functools.partial(
    pl.kernel,
    out_type=jax.ShapeDtypeStruct((_B, _D), jnp.float32),
    mesh=_mesh,
    scratch_types=[
        pltpu.VMEM((_NCH, _CHUNK), jnp.int32),
        pltpu.VMEM((_CHUNK, _D), jnp.float32),
        pltpu.VMEM((_CHUNK, _D), jnp.float32),
        pltpu.VMEM((_CHUNK, _D), jnp.float32),
        pltpu.VMEM((_CHUNK, _D), jnp.float32),
        pltpu.SemaphoreType.DMA,
        pltpu.SemaphoreType.DMA,
        pltpu.SemaphoreType.DMA,
        pltpu.SemaphoreType.DMA,
    ],
)
def _embed_sc(tok_hbm, table_hbm, out_hbm, idx_v, g0, g1, p0, p1,
              gs0, gs1, ps0, ps1):
    wid = lax.axis_index("s") * _NC + lax.axis_index("c")
    base = wid * _BPW
    gbufs = (g0, g1)
    pbufs = (p0, p1)
    gsems = (gs0, gs1)
    psems = (ps0, ps1)

    pltpu.sync_copy(tok_hbm.at[wid], idx_v)

    def start_gather(c, b):
        pltpu.async_copy(table_hbm.at[idx_v.at[c]], gbufs[b], gsems[b])

    def wait_gather(b):
        pltpu.make_async_copy(table_hbm.at[idx_v.at[0]], gbufs[b],
                              gsems[b]).wait()

    def out_slab(c):
        return out_hbm.at[pl.ds(base + c * _CHUNK, _CHUNK)]

    def start_put(c, b):
        pltpu.async_copy(pbufs[b], out_slab(c), psems[b])

    def wait_put(b):
        pltpu.make_async_copy(pbufs[b], out_slab(0), psems[b]).wait()

    start_gather(0, 0)
    start_gather(1, 1)

    @pl.loop(0, _NCH, step=2)
    def _chunks(c0):
        for b in range(2):
            c = c0 + b
            wait_gather(b)

            @pl.when(c >= 2)
            def _():
                wait_put(b)

            @pl.loop(0, _CHUNK)
            def _scale(r):
                for j in range(_D // _L):
                    sl = pl.ds(j * _L, _L)
                    pbufs[b][r, sl] = gbufs[b][r, sl] * _SCALE

            start_put(c, b)

            @pl.when(c + 2 < _NCH)
            def _():
                start_gather(c + 2, b)

    wait_put(0)
    wait_put(1)


def kernel(tokens, table):
    tok = tokens.reshape(_NW, _NCH, _CHUNK).astype(jnp.int32)
    out = _embed_sc(tok, table)
    return out.reshape(tokens.shape + (_D,))
